# Initial kernel scaffold; baseline (speedup 1.0000x reference)
import jax
import jax.numpy as jnp
from jax.experimental import pallas as pl


def kernel(inputs, token_table, pos_table):
    raise NotImplementedError("write your pallas kernel here")


# speedup vs baseline: 1.3905x; 1.3905x over previous
import functools

import jax
import jax.numpy as jnp
from jax import lax
from jax.experimental import pallas as pl
from jax.experimental.pallas import tpu as pltpu
from jax.experimental.pallas import tpu_sc as plsc

MAXLEN = 200
EMBED_DIM = 32
BATCH = 4096

NC, NS, L = 2, 16, 16
NW = NC * NS
ROWS_PER_CHUNK = 4
IDX_PER_CHUNK = ROWS_PER_CHUNK * MAXLEN
GATHERS = 8
IDX_PER_GATHER = IDX_PER_CHUNK // GATHERS
CHUNKS_PER_W = BATCH // (NW * ROWS_PER_CHUNK)
N_CHUNKS = BATCH // ROWS_PER_CHUNK


def _body(idx_hbm, table_hbm, pos_hbm, out_hbm, idx_v, rows_v, pos_v, gsem):
    wid = lax.axis_index("s") * NC + lax.axis_index("c")

    pltpu.sync_copy(pos_hbm, pos_v)

    @pl.loop(0, CHUNKS_PER_W)
    def _chunk(t):
        c = wid * CHUNKS_PER_W + t

        pltpu.sync_copy(idx_hbm.at[c], idx_v)

        copies = []
        for j in range(GATHERS):
            dst = rows_v.at[pl.ds(j * IDX_PER_GATHER, IDX_PER_GATHER), :]
            copies.append(pltpu.async_copy(table_hbm.at[idx_v.at[j]], dst, gsem))
        for cp in copies:
            cp.wait()

        @pl.loop(0, MAXLEN)
        def _pos(p):
            pv0 = pos_v[p, pl.ds(0, L)]
            pv1 = pos_v[p, pl.ds(L, L)]
            for r in range(ROWS_PER_CHUNK):
                row = r * MAXLEN + p
                plsc.addupdate(rows_v.at[row, pl.ds(0, L)], pv0)
                plsc.addupdate(rows_v.at[row, pl.ds(L, L)], pv1)

        pltpu.sync_copy(rows_v, out_hbm.at[pl.ds(c * IDX_PER_CHUNK, IDX_PER_CHUNK), :])


@jax.jit
def _embed(inputs, token_table, pos_table):
    idx = inputs.reshape(N_CHUNKS, GATHERS, IDX_PER_GATHER).astype(jnp.int32)
    mesh = plsc.VectorSubcoreMesh(core_axis_name="c", subcore_axis_name="s")
    out = pl.kernel(
        _body,
        out_type=jax.ShapeDtypeStruct((BATCH * MAXLEN, EMBED_DIM), jnp.float32),
        mesh=mesh,
        scratch_types=[
            pltpu.VMEM((GATHERS, IDX_PER_GATHER), jnp.int32),
            pltpu.VMEM((IDX_PER_CHUNK, EMBED_DIM), jnp.float32),
            pltpu.VMEM((MAXLEN, EMBED_DIM), jnp.float32),
            pltpu.SemaphoreType.DMA,
        ],
        compiler_params=pltpu.CompilerParams(use_tc_tiling_on_sc=False),
    )(idx, token_table, pos_table)
    return out.reshape(BATCH, MAXLEN, EMBED_DIM)


def kernel(inputs, token_table, pos_table):
    return _embed(inputs, token_table, pos_table)


# speedup vs baseline: 1.4612x; 1.0508x over previous
import functools

import jax
import jax.numpy as jnp
from jax import lax
from jax.experimental import pallas as pl
from jax.experimental.pallas import tpu as pltpu
from jax.experimental.pallas import tpu_sc as plsc

MAXLEN = 200
EMBED_DIM = 32
BATCH = 4096

NC, NS, L = 2, 16, 16
NW = NC * NS
ROWS_PER_CHUNK = 4
IDX_PER_CHUNK = ROWS_PER_CHUNK * MAXLEN
GATHERS = 8
IDX_PER_GATHER = IDX_PER_CHUNK // GATHERS
CHUNKS_PER_W = BATCH // (NW * ROWS_PER_CHUNK)
N_CHUNKS = BATCH // ROWS_PER_CHUNK


def _body(idx_hbm, table_hbm, pos_hbm, out_hbm, idx_v, rows_v, pos_v, gsems):
    wid = lax.axis_index("s") * NC + lax.axis_index("c")

    pltpu.sync_copy(pos_hbm, pos_v)

    def fire(b, c):
        pltpu.sync_copy(idx_hbm.at[c], idx_v.at[b])
        for j in range(GATHERS):
            dst = rows_v.at[b, pl.ds(j * IDX_PER_GATHER, IDX_PER_GATHER), :]
            pltpu.async_copy(table_hbm.at[idx_v.at[b, j]], dst, gsems[b])

    def consume(b, c):
        pltpu.make_async_copy(
            out_hbm.at[pl.ds(0, IDX_PER_CHUNK), :], rows_v.at[b], gsems[b]
        ).wait()

        @pl.loop(0, MAXLEN)
        def _pos(p):
            pv0 = pos_v[p, pl.ds(0, L)]
            pv1 = pos_v[p, pl.ds(L, L)]
            for r in range(ROWS_PER_CHUNK):
                row = r * MAXLEN + p
                plsc.addupdate(rows_v.at[b, row, pl.ds(0, L)], pv0)
                plsc.addupdate(rows_v.at[b, row, pl.ds(L, L)], pv1)

        pltpu.sync_copy(
            rows_v.at[b], out_hbm.at[pl.ds(c * IDX_PER_CHUNK, IDX_PER_CHUNK), :]
        )

    c0 = wid * CHUNKS_PER_W
    fire(0, c0)
    fire(1, c0 + 1)

    @pl.loop(0, CHUNKS_PER_W, step=2)
    def _chunk(t):
        consume(0, c0 + t)

        @pl.when(t + 2 < CHUNKS_PER_W)
        def _():
            fire(0, c0 + t + 2)

        consume(1, c0 + t + 1)

        @pl.when(t + 3 < CHUNKS_PER_W)
        def _():
            fire(1, c0 + t + 3)


@jax.jit
def _embed(inputs, token_table, pos_table):
    idx = inputs.reshape(N_CHUNKS, GATHERS, IDX_PER_GATHER).astype(jnp.int32)
    mesh = plsc.VectorSubcoreMesh(core_axis_name="c", subcore_axis_name="s")
    out = pl.kernel(
        _body,
        out_type=jax.ShapeDtypeStruct((BATCH * MAXLEN, EMBED_DIM), jnp.float32),
        mesh=mesh,
        scratch_types=[
            pltpu.VMEM((2, GATHERS, IDX_PER_GATHER), jnp.int32),
            pltpu.VMEM((2, IDX_PER_CHUNK, EMBED_DIM), jnp.float32),
            pltpu.VMEM((MAXLEN, EMBED_DIM), jnp.float32),
            [pltpu.SemaphoreType.DMA, pltpu.SemaphoreType.DMA],
        ],
        compiler_params=pltpu.CompilerParams(use_tc_tiling_on_sc=False),
    )(idx, token_table, pos_table)
    return out.reshape(BATCH, MAXLEN, EMBED_DIM)


def kernel(inputs, token_table, pos_table):
    return _embed(inputs, token_table, pos_table)


# speedup vs baseline: 1.4661x; 1.0034x over previous
import functools

import jax
import jax.numpy as jnp
from jax import lax
from jax.experimental import pallas as pl
from jax.experimental.pallas import tpu as pltpu
from jax.experimental.pallas import tpu_sc as plsc

MAXLEN = 200
EMBED_DIM = 32
BATCH = 4096

NC, NS, L = 2, 16, 16
NW = NC * NS
ROWS_PER_CHUNK = 4
IDX_PER_CHUNK = ROWS_PER_CHUNK * MAXLEN
SPLIT_OFFS = (0, 96)
SPLIT_LENS = (96, 104)
GATHERS = ROWS_PER_CHUNK * len(SPLIT_OFFS)
CHUNKS_PER_W = BATCH // (NW * ROWS_PER_CHUNK)


def _body(idx_hbm, table_hbm, pos_hbm, out_hbm, idx_v, rows_v, pos_v, gsems):
    wid = lax.axis_index("s") * NC + lax.axis_index("c")

    pltpu.sync_copy(pos_hbm, pos_v)

    def fire(b, c):
        r0 = c * ROWS_PER_CHUNK
        pltpu.sync_copy(idx_hbm.at[pl.ds(r0, ROWS_PER_CHUNK), :], idx_v.at[b])
        for r in range(ROWS_PER_CHUNK):
            for off, ln in zip(SPLIT_OFFS, SPLIT_LENS):
                idx = idx_v.at[b, r, pl.ds(off, ln)]
                dst = rows_v.at[b, r, pl.ds(off, ln), :]
                pltpu.async_copy(table_hbm.at[idx], dst, gsems[b])

    def consume(b, c):
        r0 = c * ROWS_PER_CHUNK
        pltpu.make_async_copy(
            out_hbm.at[pl.ds(0, ROWS_PER_CHUNK), :, :], rows_v.at[b], gsems[b]
        ).wait()

        @pl.loop(0, MAXLEN)
        def _pos(p):
            pv0 = pos_v[p, pl.ds(0, L)]
            pv1 = pos_v[p, pl.ds(L, L)]
            for r in range(ROWS_PER_CHUNK):
                plsc.addupdate(rows_v.at[b, r, p, pl.ds(0, L)], pv0)
                plsc.addupdate(rows_v.at[b, r, p, pl.ds(L, L)], pv1)

        pltpu.sync_copy(
            rows_v.at[b], out_hbm.at[pl.ds(r0, ROWS_PER_CHUNK), :, :]
        )

    c0 = wid * CHUNKS_PER_W
    fire(0, c0)
    fire(1, c0 + 1)

    @pl.loop(0, CHUNKS_PER_W, step=2)
    def _chunk(t):
        consume(0, c0 + t)

        @pl.when(t + 2 < CHUNKS_PER_W)
        def _():
            fire(0, c0 + t + 2)

        consume(1, c0 + t + 1)

        @pl.when(t + 3 < CHUNKS_PER_W)
        def _():
            fire(1, c0 + t + 3)


@jax.jit
def _embed(inputs, token_table, pos_table):
    mesh = plsc.VectorSubcoreMesh(core_axis_name="c", subcore_axis_name="s")
    return pl.kernel(
        _body,
        out_type=jax.ShapeDtypeStruct((BATCH, MAXLEN, EMBED_DIM), jnp.float32),
        mesh=mesh,
        scratch_types=[
            pltpu.VMEM((2, ROWS_PER_CHUNK, MAXLEN), jnp.int32),
            pltpu.VMEM((2, ROWS_PER_CHUNK, MAXLEN, EMBED_DIM), jnp.float32),
            pltpu.VMEM((MAXLEN, EMBED_DIM), jnp.float32),
            [pltpu.SemaphoreType.DMA, pltpu.SemaphoreType.DMA],
        ],
        compiler_params=pltpu.CompilerParams(use_tc_tiling_on_sc=False),
    )(inputs, token_table, pos_table)


def kernel(inputs, token_table, pos_table):
    return _embed(inputs, token_table, pos_table)
